# baseline (speedup 1.0000x reference)
import functools

import jax
import jax.numpy as jnp
from jax.experimental import pallas as pl
from jax.experimental.pallas import tpu as pltpu

B, N, T, C_IN, C_SP, C_OUT = 8, 1024, 16, 32, 16, 32
T1 = T - 2
T2 = T1 - 2
BN_COUNT = B * T2 * C_OUT
EPS = 1e-5


def _band_mask(t_in, t_out):
    t = jnp.arange(t_in)[:, None, None]
    tp = jnp.arange(t_out)[None, :, None]
    k = jnp.arange(3)[None, None, :]
    return (t == tp + k).astype(jnp.float32)


def _conv_weight_2d(w, t_in, t_out):
    m = _band_mask(t_in, t_out)
    wk = w[:, :, 0, :]
    big = jnp.einsum('tpk,ock->tcpo', m, wk)
    return big.reshape(t_in * wk.shape[1], t_out * wk.shape[0])


def _theta_blockdiag(theta, t_len):
    eye = jnp.eye(t_len, dtype=jnp.float32)
    big = jnp.einsum('pq,cs->pcqs', eye, theta)
    return big.reshape(t_len * theta.shape[0], t_len * theta.shape[1])


def _stgcn_body(x_ref, a_ref, w1_ref, w2_ref, w3_ref, b1_ref, b2_ref, b3_ref,
                th_ref, v1_ref, v2_ref, v3_ref, c1_ref, c2_ref, c3_ref,
                g_ref, be_ref, out_ref, t3_ref, s1_ref, s2_ref):
    i = pl.program_id(0)

    @pl.when(i < B)
    def _compute():
        b = i
        x = x_ref[0]

        z1 = jnp.dot(x, w1_ref[...], preferred_element_type=jnp.float32) + b1_ref[...]
        z2 = jnp.dot(x, w2_ref[...], preferred_element_type=jnp.float32) + b2_ref[...]
        z3 = jnp.dot(x, w3_ref[...], preferred_element_type=jnp.float32) + b3_ref[...]
        sig = 1.0 / (1.0 + jnp.exp(-z2))
        t_feat = jnp.maximum(z1 + sig + z3, 0.0)

        u = jnp.dot(t_feat, th_ref[...], preferred_element_type=jnp.float32)
        m = jnp.dot(a_ref[...], u, preferred_element_type=jnp.float32)
        t2 = jnp.maximum(m, 0.0)

        y1 = jnp.dot(t2, v1_ref[...], preferred_element_type=jnp.float32) + c1_ref[...]
        y2 = jnp.dot(t2, v2_ref[...], preferred_element_type=jnp.float32) + c2_ref[...]
        y3 = jnp.dot(t2, v3_ref[...], preferred_element_type=jnp.float32) + c3_ref[...]
        sig2 = 1.0 / (1.0 + jnp.exp(-y2))
        t3 = jnp.maximum(y1 + sig2 + y3, 0.0)

        t3_ref[b] = t3
        rs = jnp.sum(t3, axis=1, keepdims=True)
        rq = jnp.sum(t3 * t3, axis=1, keepdims=True)

        @pl.when(b == 0)
        def _():
            s1_ref[...] = rs
            s2_ref[...] = rq

        @pl.when(b > 0)
        def _():
            s1_ref[...] = s1_ref[...] + rs
            s2_ref[...] = s2_ref[...] + rq

    @pl.when(i >= B)
    def _normalize():
        bb = i - B
        inv_n = 1.0 / BN_COUNT
        mean = s1_ref[...] * inv_n
        var = s2_ref[...] * inv_n - mean * mean
        scale = g_ref[...] * jax.lax.rsqrt(var + EPS)
        shift = be_ref[...] - mean * scale
        y = t3_ref[bb] * scale + shift
        for t in range(T2):
            out_ref[0, :, t, :] = y[:, 32 * t:32 * (t + 1)]


@functools.partial(jax.jit, static_argnames=())
def kernel(X, A_hat, t1_w1, t1_b1, t1_w2, t1_b2, t1_w3, t1_b3, Theta1,
           t2_w1, t2_b1, t2_w2, t2_b2, t2_w3, t2_b3, bn_gamma, bn_beta):
    w1 = _conv_weight_2d(t1_w1, T, T1)
    w2 = _conv_weight_2d(t1_w2, T, T1)
    w3 = _conv_weight_2d(t1_w3, T, T1)
    b1 = jnp.tile(t1_b1, T1)[None, :]
    b2 = jnp.tile(t1_b2, T1)[None, :]
    b3 = jnp.tile(t1_b3, T1)[None, :]
    th = _theta_blockdiag(Theta1, T1)
    v1 = _conv_weight_2d(t2_w1, T1, T2)
    v2 = _conv_weight_2d(t2_w2, T1, T2)
    v3 = _conv_weight_2d(t2_w3, T1, T2)
    c1 = jnp.tile(t2_b1, T2)[None, :]
    c2 = jnp.tile(t2_b2, T2)[None, :]
    c3 = jnp.tile(t2_b3, T2)[None, :]
    x2 = X.reshape(B, N, T * C_IN)
    g = bn_gamma.reshape(N, 1)
    be = bn_beta.reshape(N, 1)

    full = lambda shape: pl.BlockSpec(shape, lambda i: (0,) * len(shape))
    out = pl.pallas_call(
        _stgcn_body,
        grid=(2 * B,),
        in_specs=[
            pl.BlockSpec((1, N, T * C_IN), lambda i: (jnp.minimum(i, B - 1), 0, 0)),
            full((N, N)),
            full((T * C_IN, T1 * C_OUT)),
            full((T * C_IN, T1 * C_OUT)),
            full((T * C_IN, T1 * C_OUT)),
            full((1, T1 * C_OUT)),
            full((1, T1 * C_OUT)),
            full((1, T1 * C_OUT)),
            full((T1 * C_OUT, T1 * C_SP)),
            full((T1 * C_SP, T2 * C_OUT)),
            full((T1 * C_SP, T2 * C_OUT)),
            full((T1 * C_SP, T2 * C_OUT)),
            full((1, T2 * C_OUT)),
            full((1, T2 * C_OUT)),
            full((1, T2 * C_OUT)),
            full((N, 1)),
            full((N, 1)),
        ],
        out_specs=pl.BlockSpec((1, N, T2, C_OUT),
                               lambda i: (jnp.maximum(i - B, 0), 0, 0, 0)),
        out_shape=jax.ShapeDtypeStruct((B, N, T2, C_OUT), jnp.float32),
        scratch_shapes=[
            pltpu.VMEM((B, N, T2 * C_OUT), jnp.float32),
            pltpu.VMEM((N, 1), jnp.float32),
            pltpu.VMEM((N, 1), jnp.float32),
        ],
    )(x2, A_hat, w1, w2, w3, b1, b2, b3, th, v1, v2, v3, c1, c2, c3, g, be)
    return out


# speedup vs baseline: 2.9428x; 2.9428x over previous
import functools

import jax
import jax.numpy as jnp
from jax.experimental import pallas as pl
from jax.experimental.pallas import tpu as pltpu

B, N, T, C_IN, C_SP, C_OUT = 8, 1024, 16, 32, 16, 32
T1 = T - 2
T2 = T1 - 2
BN_COUNT = B * T2 * C_OUT
EPS = 1e-5


def _band_mask(t_in, t_out):
    t = jnp.arange(t_in)[:, None, None]
    tp = jnp.arange(t_out)[None, :, None]
    k = jnp.arange(3)[None, None, :]
    return (t == tp + k).astype(jnp.float32)


def _conv_weight_2d_t(w, t_in, t_out):
    m = _band_mask(t_in, t_out)
    wk = w[:, :, 0, :]
    big = jnp.einsum('tpk,ock->potc', m, wk)
    return big.reshape(t_out * wk.shape[0], t_in * wk.shape[1])


def _theta_blockdiag_t(theta, t_len):
    eye = jnp.eye(t_len, dtype=jnp.float32)
    big = jnp.einsum('pq,cs->qspc', eye, theta)
    return big.reshape(t_len * theta.shape[1], t_len * theta.shape[0])


def _stgcn_body(x_ref, at_ref, w1_ref, w2_ref, w3_ref, b1_ref, b2_ref, b3_ref,
                th_ref, v1_ref, v2_ref, v3_ref, c1_ref, c2_ref, c3_ref,
                g_ref, be_ref, out_ref, t3_ref, s1_ref, s2_ref):
    b = pl.program_id(0)
    x = x_ref[0]

    z1 = jnp.dot(w1_ref[...], x, preferred_element_type=jnp.float32) + b1_ref[...]
    z2 = jnp.dot(w2_ref[...], x, preferred_element_type=jnp.float32) + b2_ref[...]
    z3 = jnp.dot(w3_ref[...], x, preferred_element_type=jnp.float32) + b3_ref[...]
    sig = 1.0 / (1.0 + jnp.exp(-z2))
    t_feat = jnp.maximum(z1 + sig + z3, 0.0)

    u = jnp.dot(th_ref[...], t_feat, preferred_element_type=jnp.float32)
    m = jnp.dot(u, at_ref[...], preferred_element_type=jnp.float32)
    t2 = jnp.maximum(m, 0.0)

    y1 = jnp.dot(v1_ref[...], t2, preferred_element_type=jnp.float32) + c1_ref[...]
    y2 = jnp.dot(v2_ref[...], t2, preferred_element_type=jnp.float32) + c2_ref[...]
    y3 = jnp.dot(v3_ref[...], t2, preferred_element_type=jnp.float32) + c3_ref[...]
    sig2 = 1.0 / (1.0 + jnp.exp(-y2))
    t3 = jnp.maximum(y1 + sig2 + y3, 0.0)

    t3_ref[b] = t3
    rs = jnp.sum(t3, axis=0, keepdims=True)
    rq = jnp.sum(t3 * t3, axis=0, keepdims=True)

    @pl.when(b == 0)
    def _():
        s1_ref[...] = rs
        s2_ref[...] = rq

    @pl.when(b > 0)
    def _():
        s1_ref[...] = s1_ref[...] + rs
        s2_ref[...] = s2_ref[...] + rq

    @pl.when(b == B - 1)
    def _():
        inv_n = 1.0 / BN_COUNT
        mean = s1_ref[...] * inv_n
        var = s2_ref[...] * inv_n - mean * mean
        scale = g_ref[...] * jax.lax.rsqrt(var + EPS)
        shift = be_ref[...] - mean * scale
        for bb in range(B):
            out_ref[bb] = t3_ref[bb] * scale + shift


@functools.partial(jax.jit, static_argnames=())
def kernel(X, A_hat, t1_w1, t1_b1, t1_w2, t1_b2, t1_w3, t1_b3, Theta1,
           t2_w1, t2_b1, t2_w2, t2_b2, t2_w3, t2_b3, bn_gamma, bn_beta):
    w1 = _conv_weight_2d_t(t1_w1, T, T1)
    w2 = _conv_weight_2d_t(t1_w2, T, T1)
    w3 = _conv_weight_2d_t(t1_w3, T, T1)
    b1 = jnp.tile(t1_b1, T1)[:, None]
    b2 = jnp.tile(t1_b2, T1)[:, None]
    b3 = jnp.tile(t1_b3, T1)[:, None]
    th = _theta_blockdiag_t(Theta1, T1)
    v1 = _conv_weight_2d_t(t2_w1, T1, T2)
    v2 = _conv_weight_2d_t(t2_w2, T1, T2)
    v3 = _conv_weight_2d_t(t2_w3, T1, T2)
    c1 = jnp.tile(t2_b1, T2)[:, None]
    c2 = jnp.tile(t2_b2, T2)[:, None]
    c3 = jnp.tile(t2_b3, T2)[:, None]
    xt = jnp.transpose(X, (0, 2, 3, 1)).reshape(B, T * C_IN, N)
    at = jnp.transpose(A_hat, (1, 0))
    g = bn_gamma[None, :]
    be = bn_beta[None, :]

    full = lambda shape: pl.BlockSpec(shape, lambda i: (0,) * len(shape))
    out = pl.pallas_call(
        _stgcn_body,
        grid=(B,),
        in_specs=[
            pl.BlockSpec((1, T * C_IN, N), lambda i: (i, 0, 0)),
            full((N, N)),
            full((T1 * C_OUT, T * C_IN)),
            full((T1 * C_OUT, T * C_IN)),
            full((T1 * C_OUT, T * C_IN)),
            full((T1 * C_OUT, 1)),
            full((T1 * C_OUT, 1)),
            full((T1 * C_OUT, 1)),
            full((T1 * C_SP, T1 * C_OUT)),
            full((T2 * C_OUT, T1 * C_SP)),
            full((T2 * C_OUT, T1 * C_SP)),
            full((T2 * C_OUT, T1 * C_SP)),
            full((T2 * C_OUT, 1)),
            full((T2 * C_OUT, 1)),
            full((T2 * C_OUT, 1)),
            full((1, N)),
            full((1, N)),
        ],
        out_specs=full((B, T2 * C_OUT, N)),
        out_shape=jax.ShapeDtypeStruct((B, T2 * C_OUT, N), jnp.float32),
        scratch_shapes=[
            pltpu.VMEM((B, T2 * C_OUT, N), jnp.float32),
            pltpu.VMEM((1, N), jnp.float32),
            pltpu.VMEM((1, N), jnp.float32),
        ],
    )(xt, at, w1, w2, w3, b1, b2, b3, th, v1, v2, v3, c1, c2, c3, g, be)
    return jnp.transpose(out.reshape(B, T2, C_OUT, N), (0, 3, 1, 2))


# speedup vs baseline: 3.0338x; 1.0309x over previous
import functools

import jax
import jax.numpy as jnp
from jax.experimental import pallas as pl
from jax.experimental.pallas import tpu as pltpu

B, N, T, C_IN, C_SP, C_OUT = 8, 1024, 16, 32, 16, 32
T1 = T - 2
T2 = T1 - 2
BN_COUNT = B * T2 * C_OUT
EPS = 1e-5


def _band_mask(t_in, t_out):
    t = jnp.arange(t_in)[:, None, None]
    tp = jnp.arange(t_out)[None, :, None]
    k = jnp.arange(3)[None, None, :]
    return (t == tp + k).astype(jnp.float32)


def _conv_weight_2d_t(w, t_in, t_out):
    m = _band_mask(t_in, t_out)
    wk = w[:, :, 0, :]
    big = jnp.einsum('tpk,ock->potc', m, wk)
    return big.reshape(t_out * wk.shape[0], t_in * wk.shape[1])


def _theta_blockdiag_t(theta, t_len):
    eye = jnp.eye(t_len, dtype=jnp.float32)
    big = jnp.einsum('pq,cs->qspc', eye, theta)
    return big.reshape(t_len * theta.shape[1], t_len * theta.shape[0])


def _stgcn_body(x_ref, at_ref, w1_ref, w2_ref, w3_ref, b1_ref, b2_ref, b3_ref,
                th_ref, v1_ref, v2_ref, v3_ref, c1_ref, c2_ref, c3_ref,
                g_ref, be_ref, out_ref, t3_ref, s1_ref, s2_ref):
    b = pl.program_id(0)
    x = x_ref[0].astype(jnp.bfloat16)

    z1 = jnp.dot(w1_ref[...], x, preferred_element_type=jnp.float32) + b1_ref[...]
    z2 = jnp.dot(w2_ref[...], x, preferred_element_type=jnp.float32) + b2_ref[...]
    z3 = jnp.dot(w3_ref[...], x, preferred_element_type=jnp.float32) + b3_ref[...]
    sig = 1.0 / (1.0 + jnp.exp(-z2))
    t_feat = jnp.maximum(z1 + sig + z3, 0.0).astype(jnp.bfloat16)

    u = jnp.dot(th_ref[...], t_feat, preferred_element_type=jnp.float32)
    m = jax.lax.dot_general(u.astype(jnp.bfloat16), at_ref[...],
                            (((1,), (1,)), ((), ())),
                            preferred_element_type=jnp.float32)
    t2 = jnp.maximum(m, 0.0).astype(jnp.bfloat16)

    y1 = jnp.dot(v1_ref[...], t2, preferred_element_type=jnp.float32) + c1_ref[...]
    y2 = jnp.dot(v2_ref[...], t2, preferred_element_type=jnp.float32) + c2_ref[...]
    y3 = jnp.dot(v3_ref[...], t2, preferred_element_type=jnp.float32) + c3_ref[...]
    sig2 = 1.0 / (1.0 + jnp.exp(-y2))
    t3 = jnp.maximum(y1 + sig2 + y3, 0.0)

    t3_ref[b] = t3
    rs = jnp.sum(t3, axis=0, keepdims=True)
    rq = jnp.sum(t3 * t3, axis=0, keepdims=True)

    @pl.when(b == 0)
    def _():
        s1_ref[...] = rs
        s2_ref[...] = rq

    @pl.when(b > 0)
    def _():
        s1_ref[...] = s1_ref[...] + rs
        s2_ref[...] = s2_ref[...] + rq

    @pl.when(b == B - 1)
    def _():
        inv_n = 1.0 / BN_COUNT
        mean = s1_ref[...] * inv_n
        var = s2_ref[...] * inv_n - mean * mean
        scale = g_ref[...] * jax.lax.rsqrt(var + EPS)
        shift = be_ref[...] - mean * scale
        for bb in range(B):
            out_ref[bb] = t3_ref[bb] * scale + shift


@functools.partial(jax.jit, static_argnames=())
def kernel(X, A_hat, t1_w1, t1_b1, t1_w2, t1_b2, t1_w3, t1_b3, Theta1,
           t2_w1, t2_b1, t2_w2, t2_b2, t2_w3, t2_b3, bn_gamma, bn_beta):
    bf = jnp.bfloat16
    w1 = _conv_weight_2d_t(t1_w1, T, T1).astype(bf)
    w2 = _conv_weight_2d_t(t1_w2, T, T1).astype(bf)
    w3 = _conv_weight_2d_t(t1_w3, T, T1).astype(bf)
    b1 = jnp.tile(t1_b1, T1)[:, None]
    b2 = jnp.tile(t1_b2, T1)[:, None]
    b3 = jnp.tile(t1_b3, T1)[:, None]
    th = _theta_blockdiag_t(Theta1, T1).astype(bf)
    v1 = _conv_weight_2d_t(t2_w1, T1, T2).astype(bf)
    v2 = _conv_weight_2d_t(t2_w2, T1, T2).astype(bf)
    v3 = _conv_weight_2d_t(t2_w3, T1, T2).astype(bf)
    c1 = jnp.tile(t2_b1, T2)[:, None]
    c2 = jnp.tile(t2_b2, T2)[:, None]
    c3 = jnp.tile(t2_b3, T2)[:, None]
    xt = jnp.transpose(X, (0, 2, 3, 1)).reshape(B, T * C_IN, N)
    at = A_hat.astype(bf)
    g = bn_gamma[None, :]
    be = bn_beta[None, :]

    full = lambda shape: pl.BlockSpec(shape, lambda i: (0,) * len(shape))
    out = pl.pallas_call(
        _stgcn_body,
        grid=(B,),
        in_specs=[
            pl.BlockSpec((1, T * C_IN, N), lambda i: (i, 0, 0)),
            full((N, N)),
            full((T1 * C_OUT, T * C_IN)),
            full((T1 * C_OUT, T * C_IN)),
            full((T1 * C_OUT, T * C_IN)),
            full((T1 * C_OUT, 1)),
            full((T1 * C_OUT, 1)),
            full((T1 * C_OUT, 1)),
            full((T1 * C_SP, T1 * C_OUT)),
            full((T2 * C_OUT, T1 * C_SP)),
            full((T2 * C_OUT, T1 * C_SP)),
            full((T2 * C_OUT, T1 * C_SP)),
            full((T2 * C_OUT, 1)),
            full((T2 * C_OUT, 1)),
            full((T2 * C_OUT, 1)),
            full((1, N)),
            full((1, N)),
        ],
        out_specs=full((B, T2 * C_OUT, N)),
        out_shape=jax.ShapeDtypeStruct((B, T2 * C_OUT, N), jnp.float32),
        scratch_shapes=[
            pltpu.VMEM((B, T2 * C_OUT, N), jnp.float32),
            pltpu.VMEM((1, N), jnp.float32),
            pltpu.VMEM((1, N), jnp.float32),
        ],
    )(xt, at, w1, w2, w3, b1, b2, b3, th, v1, v2, v3, c1, c2, c3, g, be)
    return jnp.transpose(out.reshape(B, T2, C_OUT, N), (0, 3, 1, 2))


# speedup vs baseline: 3.6736x; 1.2109x over previous
import functools

import jax
import jax.numpy as jnp
from jax.experimental import pallas as pl
from jax.experimental.pallas import tpu as pltpu

B, N, T, C_IN, C_SP, C_OUT = 8, 1024, 16, 32, 16, 32
T1 = T - 2
T2 = T1 - 2
BN_COUNT = B * T2 * C_OUT
EPS = 1e-5


def _band_mask(t_in, t_out):
    t = jnp.arange(t_in)[:, None, None]
    tp = jnp.arange(t_out)[None, :, None]
    k = jnp.arange(3)[None, None, :]
    return (t == tp + k).astype(jnp.float32)


def _conv_weight_2d_t(w, t_in, t_out):
    m = _band_mask(t_in, t_out)
    wk = w[:, :, 0, :]
    big = jnp.einsum('tpk,ock->potc', m, wk)
    return big.reshape(t_out * wk.shape[0], t_in * wk.shape[1])


def _theta_blockdiag_t(theta, t_len):
    eye = jnp.eye(t_len, dtype=jnp.float32)
    big = jnp.einsum('pq,cs->qspc', eye, theta)
    return big.reshape(t_len * theta.shape[1], t_len * theta.shape[0])


def _stgcn_body(x_ref, at_ref, wc_ref, b13_ref, b2_ref,
                th_ref, vc_ref, c13_ref, c2_ref,
                g_ref, be_ref, out_ref, t3_ref, s1_ref, s2_ref):
    i = pl.program_id(0)
    H1 = T1 * C_OUT
    H2 = T2 * C_OUT

    @pl.when(i < B)
    def _compute():
        x = x_ref[0].astype(jnp.bfloat16)

        zc = jnp.dot(wc_ref[...], x, preferred_element_type=jnp.float32)
        z13 = zc[:H1] + b13_ref[...]
        z2 = zc[H1:] + b2_ref[...]
        sig = 1.0 / (1.0 + jnp.exp(-z2))
        t_feat = jnp.maximum(z13 + sig, 0.0).astype(jnp.bfloat16)

        u = jnp.dot(th_ref[...], t_feat, preferred_element_type=jnp.float32)
        m = jax.lax.dot_general(u.astype(jnp.bfloat16), at_ref[...],
                                (((1,), (1,)), ((), ())),
                                preferred_element_type=jnp.float32)
        t2 = jnp.maximum(m, 0.0).astype(jnp.bfloat16)

        yc = jnp.dot(vc_ref[...], t2, preferred_element_type=jnp.float32)
        y13 = yc[:H2] + c13_ref[...]
        y2 = yc[H2:] + c2_ref[...]
        sig2 = 1.0 / (1.0 + jnp.exp(-y2))
        t3 = jnp.maximum(y13 + sig2, 0.0)

        t3_ref[i] = t3
        rs = jnp.sum(t3, axis=0, keepdims=True)
        rq = jnp.sum(t3 * t3, axis=0, keepdims=True)

        @pl.when(i == 0)
        def _():
            s1_ref[...] = rs
            s2_ref[...] = rq

        @pl.when(i > 0)
        def _():
            s1_ref[...] = s1_ref[...] + rs
            s2_ref[...] = s2_ref[...] + rq

    @pl.when(i >= B)
    def _normalize():
        bb = i - B
        inv_n = 1.0 / BN_COUNT
        mean = s1_ref[...] * inv_n
        var = s2_ref[...] * inv_n - mean * mean
        scale = g_ref[...] * jax.lax.rsqrt(var + EPS)
        shift = be_ref[...] - mean * scale
        out_ref[0] = t3_ref[bb] * scale + shift


@functools.partial(jax.jit, static_argnames=())
def kernel(X, A_hat, t1_w1, t1_b1, t1_w2, t1_b2, t1_w3, t1_b3, Theta1,
           t2_w1, t2_b1, t2_w2, t2_b2, t2_w3, t2_b3, bn_gamma, bn_beta):
    bf = jnp.bfloat16
    wc = jnp.concatenate([
        _conv_weight_2d_t(t1_w1, T, T1) + _conv_weight_2d_t(t1_w3, T, T1),
        _conv_weight_2d_t(t1_w2, T, T1)], axis=0).astype(bf)
    b13 = jnp.tile(t1_b1 + t1_b3, T1)[:, None]
    b2 = jnp.tile(t1_b2, T1)[:, None]
    th = _theta_blockdiag_t(Theta1, T1).astype(bf)
    vc = jnp.concatenate([
        _conv_weight_2d_t(t2_w1, T1, T2) + _conv_weight_2d_t(t2_w3, T1, T2),
        _conv_weight_2d_t(t2_w2, T1, T2)], axis=0).astype(bf)
    c13 = jnp.tile(t2_b1 + t2_b3, T2)[:, None]
    c2 = jnp.tile(t2_b2, T2)[:, None]
    xt = jnp.transpose(X, (0, 2, 3, 1)).reshape(B, T * C_IN, N)
    at = A_hat.astype(bf)
    g = bn_gamma[None, :]
    be = bn_beta[None, :]

    full = lambda shape: pl.BlockSpec(shape, lambda i: (0,) * len(shape))
    out = pl.pallas_call(
        _stgcn_body,
        grid=(2 * B,),
        in_specs=[
            pl.BlockSpec((1, T * C_IN, N),
                         lambda i: (jnp.minimum(i, B - 1), 0, 0)),
            full((N, N)),
            full((2 * T1 * C_OUT, T * C_IN)),
            full((T1 * C_OUT, 1)),
            full((T1 * C_OUT, 1)),
            full((T1 * C_SP, T1 * C_OUT)),
            full((2 * T2 * C_OUT, T1 * C_SP)),
            full((T2 * C_OUT, 1)),
            full((T2 * C_OUT, 1)),
            full((1, N)),
            full((1, N)),
        ],
        out_specs=pl.BlockSpec((1, T2 * C_OUT, N),
                               lambda i: (jnp.maximum(i - B, 0), 0, 0)),
        out_shape=jax.ShapeDtypeStruct((B, T2 * C_OUT, N), jnp.float32),
        scratch_shapes=[
            pltpu.VMEM((B, T2 * C_OUT, N), jnp.float32),
            pltpu.VMEM((1, N), jnp.float32),
            pltpu.VMEM((1, N), jnp.float32),
        ],
    )(xt, at, wc, b13, b2, th, vc, c13, c2, g, be)
    return jnp.transpose(out.reshape(B, T2, C_OUT, N), (0, 3, 1, 2))
